# baseline (speedup 1.0000x reference)
import jax
import jax.numpy as jnp
from jax import lax
from jax.experimental import pallas as pl
from jax.experimental.pallas import tpu as pltpu
from jax.experimental.pallas import tpu_sc as plsc

B = 4
E = 16
C = 1024
K = E // 2
L = 16
NS = 2
NSTR = 4
BC = 128


def _sc_gate_kernel(x_hbm, out_hbm, rows_v, pmat_v, g_v):
    cid = lax.axis_index("c")
    sid = lax.axis_index("s")

    @pl.when((cid == 0) & (sid < B))
    def _():
        pltpu.sync_copy(x_hbm.at[sid], rows_v)

        def body(c, accs):
            return tuple(accs[e] + rows_v[pl.ds(e * C + c * L, L)]
                         for e in range(E))

        accs = lax.fori_loop(0, C // L, body,
                             tuple(jnp.zeros((L,), jnp.float32)
                                   for _ in range(E)))
        for e in range(E):
            pmat_v[pl.ds(e * L, L)] = accs[e]

        lane = lax.broadcasted_iota(jnp.int32, (L,), 0)
        base = lane * L
        route = jnp.zeros((L,), jnp.float32)
        for j in range(L):
            route = route + plsc.load_gather(pmat_v, [base + j])

        srt = plsc.sort_key_val(route, route, descending=True)
        if isinstance(srt, (tuple, list)):
            srt = srt[0]
        g_v[...] = jnp.where(lane < K, srt, 0.0)
        pltpu.sync_copy(g_v, out_hbm.at[sid])


def _sc_gate(x):
    mesh = plsc.VectorSubcoreMesh(core_axis_name="c", subcore_axis_name="s")
    return pl.kernel(
        _sc_gate_kernel,
        out_type=jax.ShapeDtypeStruct((B, L), jnp.float32),
        mesh=mesh,
        compiler_params=pltpu.CompilerParams(needs_layout_passes=False),
        scratch_types=[
            pltpu.VMEM((E * C,), jnp.float32),
            pltpu.VMEM((E * L,), jnp.float32),
            pltpu.VMEM((L,), jnp.float32),
        ],
    )(x.reshape(B, E * C))


def _moe_kernel(x_ref, g_ref, *refs):
    wv_refs = refs[:NSTR]
    wo_refs = refs[NSTR:2 * NSTR]
    out_ref = refs[2 * NSTR]
    i = pl.program_id(0)
    x0 = x_ref[:, 0, :]
    contrib = jnp.zeros((B, C), jnp.float32)
    for s in range(NSTR):
        v = jnp.dot(x0, wv_refs[s][0], preferred_element_type=jnp.float32)
        contrib += jnp.dot(v, wo_refs[s][0],
                           preferred_element_type=jnp.float32)

    @pl.when(i == 0)
    def _():
        out_ref[...] = contrib

    @pl.when((i > 0) & (i < NS - 1))
    def _():
        out_ref[...] += contrib

    @pl.when(i == NS - 1)
    def _():
        g = jnp.sum(g_ref[...], axis=-1, keepdims=True) * (1.0 / K)
        out_ref[...] = (out_ref[...] + contrib) * g


def kernel(x, q, Wq, Wk, Wv, Wo):
    g = _sc_gate(x)

    def wv_spec(s):
        return pl.BlockSpec((1, C, BC), lambda i, s=s: (0, 0, s * NS + i))

    def wo_spec(s):
        return pl.BlockSpec((1, BC, C), lambda i, s=s: (0, s * NS + i, 0))

    out = pl.pallas_call(
        _moe_kernel,
        grid=(NS,),
        in_specs=[pl.BlockSpec((B, E, C), lambda i: (0, 0, 0)),
                  pl.BlockSpec((B, L), lambda i: (0, 0))]
        + [wv_spec(s) for s in range(NSTR)]
        + [wo_spec(s) for s in range(NSTR)],
        out_specs=pl.BlockSpec((B, C), lambda i: (0, 0)),
        out_shape=jax.ShapeDtypeStruct((B, C), jnp.float32),
    )(x, g, *([Wv] * NSTR), *([Wo] * NSTR))
    return out[:, None, :]


# speedup vs baseline: 1.1965x; 1.1965x over previous
import jax
import jax.numpy as jnp
from jax import lax
from jax.experimental import pallas as pl
from jax.experimental.pallas import tpu as pltpu
from jax.experimental.pallas import tpu_sc as plsc

B = 4
E = 16
C = 1024
K = E // 2
L = 16
NS = 2
NSTR = 4
BC = 128


def _sc_gate_kernel(x_hbm, out_hbm, rows_v, pmat_v, g_v):
    cid = lax.axis_index("c")
    sid = lax.axis_index("s")

    @pl.when((cid == 0) & (sid < B))
    def _():
        pltpu.sync_copy(x_hbm.at[sid], rows_v)

        def body(c, accs):
            return tuple(accs[e] + rows_v[pl.ds(e * C + c * L, L)]
                         for e in range(E))

        accs = lax.fori_loop(0, C // L, body,
                             tuple(jnp.zeros((L,), jnp.float32)
                                   for _ in range(E)))
        for e in range(E):
            pmat_v[pl.ds(e * L, L)] = accs[e]

        lane = lax.broadcasted_iota(jnp.int32, (L,), 0)
        base = lane * L
        route = jnp.zeros((L,), jnp.float32)
        for j in range(L):
            route = route + plsc.load_gather(pmat_v, [base + j])

        srt = plsc.sort_key_val(route, route, descending=True)
        if isinstance(srt, (tuple, list)):
            srt = srt[0]
        g_v[...] = jnp.where(lane < K, srt, 0.0)
        pltpu.sync_copy(g_v, out_hbm.at[sid])


def _sc_gate(x):
    mesh = plsc.VectorSubcoreMesh(core_axis_name="c", subcore_axis_name="s",
                                  num_cores=1)
    return pl.kernel(
        _sc_gate_kernel,
        out_type=jax.ShapeDtypeStruct((B, L), jnp.float32),
        mesh=mesh,
        compiler_params=pltpu.CompilerParams(needs_layout_passes=False),
        scratch_types=[
            pltpu.VMEM((E * C,), jnp.float32),
            pltpu.VMEM((E * L,), jnp.float32),
            pltpu.VMEM((L,), jnp.float32),
        ],
    )(x.reshape(B, E * C))


def _moe_kernel(x_ref, *refs):
    wv_refs = refs[:NSTR]
    wo_refs = refs[NSTR:2 * NSTR]
    out_ref = refs[2 * NSTR]
    i = pl.program_id(0)
    x0 = x_ref[:, 0, :]
    contrib = jnp.zeros((B, C), jnp.float32)
    for s in range(NSTR):
        v = jnp.dot(x0, wv_refs[s][0], preferred_element_type=jnp.float32)
        contrib += jnp.dot(v, wo_refs[s][0],
                           preferred_element_type=jnp.float32)

    @pl.when(i == 0)
    def _():
        out_ref[...] = contrib

    @pl.when(i > 0)
    def _():
        out_ref[...] += contrib


def _scale_kernel(o_ref, g_ref, out_ref):
    g = jnp.sum(g_ref[...], axis=-1, keepdims=True) * (1.0 / K)
    out_ref[...] = o_ref[...] * g


def kernel(x, q, Wq, Wk, Wv, Wo):
    g = _sc_gate(x)

    def wv_spec(s):
        return pl.BlockSpec((1, C, BC), lambda i, s=s: (0, 0, s * NS + i))

    def wo_spec(s):
        return pl.BlockSpec((1, BC, C), lambda i, s=s: (0, s * NS + i, 0))

    o = pl.pallas_call(
        _moe_kernel,
        grid=(NS,),
        in_specs=[pl.BlockSpec((B, E, C), lambda i: (0, 0, 0))]
        + [wv_spec(s) for s in range(NSTR)]
        + [wo_spec(s) for s in range(NSTR)],
        out_specs=pl.BlockSpec((B, C), lambda i: (0, 0)),
        out_shape=jax.ShapeDtypeStruct((B, C), jnp.float32),
    )(x, *([Wv] * NSTR), *([Wo] * NSTR))

    out = pl.pallas_call(
        _scale_kernel,
        out_shape=jax.ShapeDtypeStruct((B, C), jnp.float32),
    )(o, g)
    return out[:, None, :]
